# baseline (speedup 1.0000x reference)
import jax
import jax.numpy as jnp
from jax.experimental import pallas as pl
from jax.experimental.pallas import tpu as pltpu

_TOKENS = 32768
_DIM = 768
_EXPERTS = 64
_NSLAB = 4
_NW = 4
_WROWS = _TOKENS // (_NSLAB * _NW)
_KIN = 2
_KOUT = 4


def _router_kernel(x_ref, c_ref, t_ref, out_ref, xbuf, obuf, insem, outsem):
    c = c_ref[:]
    c_ss = jnp.sum(c * c, axis=1, keepdims=True)
    cn = c * jax.lax.rsqrt(jnp.maximum(c_ss, 1e-24))
    inv_t = 1.0 / t_ref[0]

    def in_copy(w):
        return pltpu.make_async_copy(
            x_ref.at[:, pl.ds(w * _WROWS, _WROWS), :],
            xbuf.at[w % _KIN],
            insem.at[w % _KIN],
        )

    def out_copy(w, s, k):
        return pltpu.make_async_copy(
            obuf.at[k % _KOUT],
            out_ref.at[pl.ds((s * _NW + w) * _WROWS, _WROWS), :],
            outsem.at[k % _KOUT],
        )

    for w in range(_KIN):
        in_copy(w).start()

    k = 0
    pending = []
    for w in range(_NW):
        in_copy(w).wait()
        for s in range(_NSLAB):
            xb = xbuf[w % _KIN, s]
            x_ss = jnp.sum(xb * xb, axis=1, keepdims=True)
            inv_norm = jax.lax.rsqrt(jnp.maximum(x_ss, 1e-24))
            logits = jax.lax.dot_general(
                xb, cn, (((1,), (1,)), ((), ())),
                preferred_element_type=jnp.float32,
            )
            if k >= _KOUT:
                pending[k - _KOUT].wait()
            obuf[k % _KOUT] = logits * (inv_norm * inv_t)
            cp = out_copy(w, s, k)
            cp.start()
            pending.append(cp)
            k += 1
        if w + _KIN < _NW:
            in_copy(w + _KIN).start()

    for j in range(k - _KOUT, k):
        pending[j].wait()


@jax.jit
def kernel(x, centroids, temperature):
    x4 = x.reshape(_NSLAB, _NW * _WROWS, _DIM)
    return pl.pallas_call(
        _router_kernel,
        in_specs=[
            pl.BlockSpec(memory_space=pltpu.HBM),
            pl.BlockSpec(memory_space=pltpu.VMEM),
            pl.BlockSpec(memory_space=pltpu.SMEM),
        ],
        out_specs=pl.BlockSpec(memory_space=pltpu.HBM),
        out_shape=jax.ShapeDtypeStruct((_TOKENS, _EXPERTS), jnp.float32),
        scratch_shapes=[
            pltpu.VMEM((_KIN, _NSLAB, _WROWS, _DIM), jnp.float32),
            pltpu.VMEM((_KOUT, _WROWS, _EXPERTS), jnp.float32),
            pltpu.SemaphoreType.DMA((_KIN,)),
            pltpu.SemaphoreType.DMA((_KOUT,)),
        ],
        compiler_params=pltpu.CompilerParams(
            vmem_limit_bytes=100 * 1024 * 1024,
        ),
    )(x4, centroids, temperature)


# speedup vs baseline: 1.1018x; 1.1018x over previous
import jax
import jax.numpy as jnp
from jax.experimental import pallas as pl
from jax.experimental.pallas import tpu as pltpu

_TOKENS = 32768
_DIM = 768
_EXPERTS = 64
_BT = 4096


def _router_kernel(x_ref, c_ref, t_ref, out_ref, cn_ref):
    i = pl.program_id(0)

    @pl.when(i == 0)
    def _init():
        c = c_ref[:]
        c_ss = jnp.sum(c * c, axis=1, keepdims=True)
        cn_ref[:] = c * jax.lax.rsqrt(jnp.maximum(c_ss, 1e-24))

    xb = x_ref[:]
    x_ss = jnp.sum(xb * xb, axis=1, keepdims=True)
    inv_norm = jax.lax.rsqrt(jnp.maximum(x_ss, 1e-24))
    logits = jax.lax.dot_general(
        xb, cn_ref[:], (((1,), (1,)), ((), ())), preferred_element_type=jnp.float32
    )
    out_ref[pl.ds(i * _BT, _BT), :] = logits * (inv_norm / t_ref[0])


@jax.jit
def kernel(x, centroids, temperature):
    grid = (_TOKENS // _BT,)
    return pl.pallas_call(
        _router_kernel,
        grid=grid,
        in_specs=[
            pl.BlockSpec((_BT, _DIM), lambda i: (i, 0)),
            pl.BlockSpec((_EXPERTS, _DIM), lambda i: (0, 0)),
            pl.BlockSpec(memory_space=pltpu.SMEM),
        ],
        out_specs=pl.BlockSpec((_TOKENS, _EXPERTS), lambda i: (0, 0)),
        out_shape=jax.ShapeDtypeStruct((_TOKENS, _EXPERTS), jnp.float32),
        scratch_shapes=[pltpu.VMEM((_EXPERTS, _DIM), jnp.float32)],
        compiler_params=pltpu.CompilerParams(
            dimension_semantics=("arbitrary",),
            vmem_limit_bytes=100 * 1024 * 1024,
        ),
    )(x, centroids, temperature)


# speedup vs baseline: 1.1272x; 1.0230x over previous
import jax
import jax.numpy as jnp
from jax.experimental import pallas as pl
from jax.experimental.pallas import tpu as pltpu

_TOKENS = 32768
_DIM = 768
_EXPERTS = 64
_BT = 4096


def _router_kernel(x_ref, c_ref, t_ref, out_ref, cn_ref):
    @pl.when(pl.program_id(0) == 0)
    def _init():
        c = c_ref[:]
        c_ss = jnp.sum(c * c, axis=1, keepdims=True)
        cn_ref[:] = c * jax.lax.rsqrt(jnp.maximum(c_ss, 1e-24))

    xb = x_ref[:]
    x_ss = jnp.sum(xb * xb, axis=1, keepdims=True)
    inv_norm = jax.lax.rsqrt(jnp.maximum(x_ss, 1e-24))
    logits = jax.lax.dot_general(
        xb, cn_ref[:], (((1,), (1,)), ((), ())), preferred_element_type=jnp.float32
    )
    out_ref[:] = logits * (inv_norm / t_ref[0])


@jax.jit
def kernel(x, centroids, temperature):
    grid = (_TOKENS // _BT,)
    return pl.pallas_call(
        _router_kernel,
        grid=grid,
        in_specs=[
            pl.BlockSpec((_BT, _DIM), lambda i: (i, 0)),
            pl.BlockSpec((_EXPERTS, _DIM), lambda i: (0, 0)),
            pl.BlockSpec(memory_space=pltpu.SMEM),
        ],
        out_specs=pl.BlockSpec((_BT, _EXPERTS), lambda i: (i, 0)),
        out_shape=jax.ShapeDtypeStruct((_TOKENS, _EXPERTS), jnp.float32),
        scratch_shapes=[pltpu.VMEM((_EXPERTS, _DIM), jnp.float32)],
        compiler_params=pltpu.CompilerParams(
            dimension_semantics=("arbitrary",),
        ),
    )(x, centroids, temperature)
